# Initial kernel scaffold; baseline (speedup 1.0000x reference)
import jax
import jax.numpy as jnp
from jax.experimental import pallas as pl


def kernel(input, embeddings):
    raise NotImplementedError("write your pallas kernel here")


# speedup vs baseline: 8.4317x; 8.4317x over previous
import functools

import jax
import jax.numpy as jnp
from jax import lax
from jax.experimental import pallas as pl
from jax.experimental.pallas import tpu as pltpu
from jax.experimental.pallas import tpu_sc as plsc


def _rowsum_body(tbl_ref, out_ref):
    out_ref[...] = jnp.sum(tbl_ref[...], axis=1, keepdims=True)


def kernel(input, embeddings):
    V, D = embeddings.shape
    B = input.size
    idx = input.reshape(B)

    vblk = 4000
    rowsums = pl.pallas_call(
        _rowsum_body,
        grid=(V // vblk,),
        in_specs=[pl.BlockSpec((vblk, D), lambda i: (i, 0))],
        out_specs=pl.BlockSpec((vblk, 1), lambda i: (i, 0)),
        out_shape=jax.ShapeDtypeStruct((V, 1), jnp.float32),
    )(embeddings)
    rowsums = rowsums.reshape(V)

    info = plsc.get_sparse_core_info()
    NC, NS, L = info.num_cores, info.num_subcores, info.num_lanes
    NW = NC * NS
    bpw = B // NW

    @functools.partial(
        pl.kernel,
        mesh=plsc.VectorSubcoreMesh(core_axis_name="c", subcore_axis_name="s"),
        compiler_params=pltpu.CompilerParams(needs_layout_passes=False),
        out_type=jax.ShapeDtypeStruct((NW, L), jnp.float32),
        scratch_types=[
            pltpu.VMEM((V,), jnp.float32),
            pltpu.VMEM((bpw,), jnp.int32),
            pltpu.VMEM((L,), jnp.float32),
        ],
    )
    def _gather_sum(idx_hbm, rs_hbm, out_hbm, rs_v, idx_v, acc_v):
        wid = lax.axis_index("s") * NC + lax.axis_index("c")
        base = wid * bpw
        pltpu.sync_copy(rs_hbm, rs_v)
        pltpu.sync_copy(idx_hbm.at[pl.ds(base, bpw)], idx_v)
        acc_v[...] = jnp.zeros((L,), jnp.float32)

        def body(j, carry):
            iv = idx_v[pl.ds(j * L, L)]
            acc_v[...] = acc_v[...] + plsc.load_gather(rs_v, [iv])
            return carry

        lax.fori_loop(0, bpw // L, body, 0)
        pltpu.sync_copy(acc_v, out_hbm.at[wid])

    partials = _gather_sum(idx, rowsums)
    return jnp.sum(partials) / jnp.float32(B * D)


# speedup vs baseline: 8.5901x; 1.0188x over previous
import functools

import jax
import jax.numpy as jnp
from jax import lax
from jax.experimental import pallas as pl
from jax.experimental.pallas import tpu as pltpu
from jax.experimental.pallas import tpu_sc as plsc


def _rowsum_body(tbl_ref, ones_ref, out_ref):
    out_ref[...] = jnp.dot(
        tbl_ref[...], ones_ref[...], preferred_element_type=jnp.float32
    )


def kernel(input, embeddings):
    V, D = embeddings.shape
    B = input.size
    idx = input.reshape(B)

    vblk = 5000
    ones = jnp.ones((D, 1), jnp.float32)
    rowsums = pl.pallas_call(
        _rowsum_body,
        grid=(V // vblk,),
        in_specs=[
            pl.BlockSpec((vblk, D), lambda i: (i, 0)),
            pl.BlockSpec((D, 1), lambda i: (0, 0)),
        ],
        out_specs=pl.BlockSpec((vblk, 1), lambda i: (i, 0)),
        out_shape=jax.ShapeDtypeStruct((V, 1), jnp.float32),
    )(embeddings, ones)
    rowsums = rowsums.reshape(V)

    info = plsc.get_sparse_core_info()
    NC, NS, L = info.num_cores, info.num_subcores, info.num_lanes
    NW = NC * NS
    bpw = B // NW

    @functools.partial(
        pl.kernel,
        mesh=plsc.VectorSubcoreMesh(core_axis_name="c", subcore_axis_name="s"),
        compiler_params=pltpu.CompilerParams(needs_layout_passes=False),
        out_type=jax.ShapeDtypeStruct((NW, L), jnp.float32),
        scratch_types=[
            pltpu.VMEM((V,), jnp.float32),
            pltpu.VMEM((bpw,), jnp.int32),
            pltpu.VMEM((L,), jnp.float32),
        ],
    )
    def _gather_sum(idx_hbm, rs_hbm, out_hbm, rs_v, idx_v, acc_v):
        wid = lax.axis_index("s") * NC + lax.axis_index("c")
        base = wid * bpw
        pltpu.sync_copy(rs_hbm, rs_v)
        pltpu.sync_copy(idx_hbm.at[pl.ds(base, bpw)], idx_v)
        acc_v[...] = jnp.zeros((L,), jnp.float32)

        def body(j, carry):
            iv = idx_v[pl.ds(j * L, L)]
            acc_v[...] = acc_v[...] + plsc.load_gather(rs_v, [iv])
            return carry

        lax.fori_loop(0, bpw // L, body, 0)
        pltpu.sync_copy(acc_v, out_hbm.at[wid])

    partials = _gather_sum(idx, rowsums)
    return jnp.sum(partials) / jnp.float32(B * D)


# speedup vs baseline: 9.6742x; 1.1262x over previous
import functools

import jax
import jax.numpy as jnp
from jax import lax
from jax.experimental import pallas as pl
from jax.experimental.pallas import tpu as pltpu
from jax.experimental.pallas import tpu_sc as plsc

_IDX_CHUNK = 128


def _rowsum_body(tbl_ref, ones_ref, out_ref):
    out_ref[...] = jnp.dot(
        tbl_ref[...], ones_ref[...], preferred_element_type=jnp.float32
    )


def kernel(input, embeddings):
    V, D = embeddings.shape
    B = input.size

    vblk = 20000
    ones = jnp.ones((D, 1), jnp.float32)
    rowsums = pl.pallas_call(
        _rowsum_body,
        grid=(V // vblk,),
        in_specs=[
            pl.BlockSpec((vblk, D), lambda i: (i, 0)),
            pl.BlockSpec((D, 1), lambda i: (0, 0)),
        ],
        out_specs=pl.BlockSpec((vblk, 1), lambda i: (i, 0)),
        out_shape=jax.ShapeDtypeStruct((V, 1), jnp.float32),
    )(embeddings, ones)
    rowsums = rowsums.reshape(V)

    info = plsc.get_sparse_core_info()
    NC, NS, L = info.num_cores, info.num_subcores, info.num_lanes
    NW = NC * NS
    bpw = B // NW
    nchunks = bpw // _IDX_CHUNK
    idx = input.reshape(NW, nchunks, _IDX_CHUNK)

    @functools.partial(
        pl.kernel,
        mesh=plsc.VectorSubcoreMesh(core_axis_name="c", subcore_axis_name="s"),
        compiler_params=pltpu.CompilerParams(needs_layout_passes=False),
        out_type=jax.ShapeDtypeStruct((NW, L), jnp.float32),
        scratch_types=[
            pltpu.VMEM((nchunks, _IDX_CHUNK), jnp.int32),
            pltpu.VMEM((bpw,), jnp.float32),
            pltpu.VMEM((L,), jnp.float32),
            pltpu.SemaphoreType.DMA,
        ],
    )
    def _gather_sum(idx_hbm, rs_hbm, out_hbm, idx_v, vals_v, acc_v, sem):
        wid = lax.axis_index("s") * NC + lax.axis_index("c")
        pltpu.sync_copy(idx_hbm.at[wid], idx_v)

        def fire(j, carry):
            pltpu.make_async_copy(
                rs_hbm.at[idx_v.at[j]],
                vals_v.at[pl.ds(j * _IDX_CHUNK, _IDX_CHUNK)],
                sem,
            ).start()
            return carry

        lax.fori_loop(0, nchunks, fire, 0)

        def drain(j, carry):
            pltpu.make_async_copy(
                rs_hbm.at[idx_v.at[j]],
                vals_v.at[pl.ds(j * _IDX_CHUNK, _IDX_CHUNK)],
                sem,
            ).wait()
            return carry

        lax.fori_loop(0, nchunks, drain, 0)

        acc_v[...] = jnp.zeros((L,), jnp.float32)

        def body(j, carry):
            acc_v[...] = acc_v[...] + vals_v[pl.ds(j * L, L)]
            return carry

        lax.fori_loop(0, bpw // L, body, 0)
        pltpu.sync_copy(acc_v, out_hbm.at[wid])

    partials = _gather_sum(idx, rowsums)
    return jnp.sum(partials) / jnp.float32(B * D)
